# Initial kernel scaffold; baseline (speedup 1.0000x reference)
import jax
import jax.numpy as jnp
from jax.experimental import pallas as pl


def kernel(y0, y1):
    raise NotImplementedError("write your pallas kernel here")


# speedup vs baseline: 230.8151x; 230.8151x over previous
import numpy as np
import jax
import jax.numpy as jnp
from jax.experimental import pallas as pl
from jax.experimental.pallas import tpu as pltpu

H = 512
W = 512
H2 = 2 * H


def _dfb_taps():
    v = np.array([0.63, -0.193, 0.0972, -0.0526, 0.0272, -0.0144],
                 dtype=np.float32)
    f = np.concatenate((v[::-1], v))
    f[::2] = -f[::2]
    return [float(t) for t in f]


_TAPS = _dfb_taps()
_SQRT2 = float(np.sqrt(2.0))


def _circ_sep_conv(x, off):
    acc = _TAPS[0] * pltpu.roll(x, (-off) % H, 0)
    for t in range(1, 12):
        acc = acc + _TAPS[t] * pltpu.roll(x, (-(off + t)) % H, 0)
    x = acc
    acc = _TAPS[0] * pltpu.roll(x, (-off) % W, 1)
    for t in range(1, 12):
        acc = acc + _TAPS[t] * pltpu.roll(x, (-(off + t)) % W, 1)
    return acc


def _riffle(y, ir):
    for t in range(9):
        p = H2 >> t
        q = p >> 2
        up = pltpu.roll(y, H2 - q, 0)
        dn = pltpu.roll(y, q, 0)
        m = ir & (p - 1)
        y = jnp.where((m >= q) & (m < 2 * q), up,
                      jnp.where((m >= 2 * q) & (m < 3 * q), dn, y))
    return y


def _body(y0_ref, y1_ref, out_ref):
    x0 = y0_ref[0]
    a = _circ_sep_conv(x0, -5)
    p1 = (-1.0 / _SQRT2) * (y1_ref[0] + a)
    b = _circ_sep_conv(p1, -6)
    p0 = _SQRT2 * x0 + b

    s1 = pltpu.roll(p0, 0, 1, stride=1, stride_axis=0)
    s2 = pltpu.roll(p1, 1, 1, stride=1, stride_axis=0)

    y = jnp.concatenate([s1, s2], axis=0)
    ir = jax.lax.broadcasted_iota(jnp.int32, (H2, W), 0)
    y = _riffle(y, ir)

    ic = jax.lax.broadcasted_iota(jnp.int32, (H2, W), 1)
    for bit in range(10):
        rolled = pltpu.roll(y, H2 - (1 << bit), 0)
        y = jnp.where((ic & (1 << bit)) != 0, rolled, y)
    out_ref[0] = y


@jax.jit
def kernel(y0, y1):
    n, c = y0.shape[0], y0.shape[1]
    a = y0.reshape(n * c, H, W)
    b = y1.reshape(n * c, H, W)
    out = pl.pallas_call(
        _body,
        grid=(n * c,),
        in_specs=[
            pl.BlockSpec((1, H, W), lambda i: (i, 0, 0)),
            pl.BlockSpec((1, H, W), lambda i: (i, 0, 0)),
        ],
        out_specs=pl.BlockSpec((1, H2, W), lambda i: (i, 0, 0)),
        out_shape=jax.ShapeDtypeStruct((n * c, H2, W), jnp.float32),
    )(a, b)
    return out.reshape(n, c, H2, W)


# speedup vs baseline: 289.4165x; 1.2539x over previous
import numpy as np
import jax
import jax.numpy as jnp
from jax.experimental import pallas as pl
from jax.experimental.pallas import tpu as pltpu

H = 512
W = 512
H2 = 2 * H


def _dfb_taps():
    v = np.array([0.63, -0.193, 0.0972, -0.0526, 0.0272, -0.0144],
                 dtype=np.float32)
    f = np.concatenate((v[::-1], v))
    f[::2] = -f[::2]
    return [float(t) for t in f]


_TAPS = _dfb_taps()
_SQRT2 = float(np.sqrt(2.0))


def _conv_mats(off):
    cv = np.zeros((H, H), dtype=np.float32)
    r = np.arange(H)
    for t in range(12):
        cv[r, (r + off + t) % H] = _TAPS[t]
    ch = np.zeros((W, W), dtype=np.float32)
    for t in range(12):
        ch[(r + off + t) % W, r] = _TAPS[t]
    return cv, ch


_CV0, _CH0 = _conv_mats(-5)
_CV1, _CH1 = _conv_mats(-6)


def _circ_sep_conv(x, cv, ch):
    y = jax.lax.dot(cv, x, precision=jax.lax.Precision.HIGHEST,
                    preferred_element_type=jnp.float32)
    return jax.lax.dot(y, ch, precision=jax.lax.Precision.HIGHEST,
                       preferred_element_type=jnp.float32)


def _riffle(y, ir):
    for t in range(9):
        p = H2 >> t
        q = p >> 2
        up = pltpu.roll(y, H2 - q, 0)
        dn = pltpu.roll(y, q, 0)
        m = ir & (p - 1)
        y = jnp.where((m >= q) & (m < 2 * q), up,
                      jnp.where((m >= 2 * q) & (m < 3 * q), dn, y))
    return y


def _body(y0_ref, y1_ref, cv0_ref, ch0_ref, cv1_ref, ch1_ref, out_ref):
    x0 = y0_ref[0]
    a = _circ_sep_conv(x0, cv0_ref[...], ch0_ref[...])
    p1 = (-1.0 / _SQRT2) * (y1_ref[0] + a)
    b = _circ_sep_conv(p1, cv1_ref[...], ch1_ref[...])
    p0 = _SQRT2 * x0 + b

    s1 = pltpu.roll(p0, 0, 1, stride=1, stride_axis=0)
    s2 = pltpu.roll(p1, 1, 1, stride=1, stride_axis=0)

    y = jnp.concatenate([s1, s2], axis=0)
    ir = jax.lax.broadcasted_iota(jnp.int32, (H2, W), 0)
    y = _riffle(y, ir)

    ic = jax.lax.broadcasted_iota(jnp.int32, (H2, W), 1)
    for bit in range(10):
        rolled = pltpu.roll(y, H2 - (1 << bit), 0)
        y = jnp.where((ic & (1 << bit)) != 0, rolled, y)
    out_ref[0] = y


@jax.jit
def kernel(y0, y1):
    n, c = y0.shape[0], y0.shape[1]
    a = y0.reshape(n * c, H, W)
    b = y1.reshape(n * c, H, W)
    out = pl.pallas_call(
        _body,
        grid=(n * c,),
        in_specs=[
            pl.BlockSpec((1, H, W), lambda i: (i, 0, 0)),
            pl.BlockSpec((1, H, W), lambda i: (i, 0, 0)),
            pl.BlockSpec((H, H), lambda i: (0, 0)),
            pl.BlockSpec((W, W), lambda i: (0, 0)),
            pl.BlockSpec((H, H), lambda i: (0, 0)),
            pl.BlockSpec((W, W), lambda i: (0, 0)),
        ],
        out_specs=pl.BlockSpec((1, H2, W), lambda i: (i, 0, 0)),
        out_shape=jax.ShapeDtypeStruct((n * c, H2, W), jnp.float32),
    )(a, b, jnp.asarray(_CV0), jnp.asarray(_CH0),
      jnp.asarray(_CV1), jnp.asarray(_CH1))
    return out.reshape(n, c, H2, W)
